# baseline (speedup 1.0000x reference)
import functools

import jax
import jax.numpy as jnp
from jax import lax
from jax.experimental import pallas as pl
from jax.experimental.pallas import tpu as pltpu
from jax.experimental.pallas import tpu_sc as plsc

EPS = 1e-5
_NC = 2
_NS = 16
_K = 128
_DEGW = 8


def _idx_from_dst(ebuf, idx, base, n_half):
    for i in range(_K // 16):
        d = ebuf[1, pl.ds(i * 16, 16)]
        loc = d - base
        ok = (loc >= 0) & (loc < n_half)
        idx[pl.ds(i * 16, 16)] = jnp.where(ok, loc, n_half)


_KD = 512


def _make_sc_degree(e_pad_deg, n_half, rpt):
    g_cnt = e_pad_deg // (_NS * _KD)
    assert g_cnt % 4 == 0
    mesh = plsc.VectorSubcoreMesh(core_axis_name="c", subcore_axis_name="s")

    @functools.partial(
        pl.kernel,
        out_type=jax.ShapeDtypeStruct((_NC * n_half, _DEGW), jnp.float32),
        mesh=mesh,
        compiler_params=pltpu.CompilerParams(use_tc_tiling_on_sc=False),
        scratch_types=[
            pltpu.VMEM_SHARED((n_half + 8, _DEGW), jnp.float32),
            pltpu.VMEM((rpt, _DEGW), jnp.float32),
            [pltpu.VMEM((_KD,), jnp.int32)] * 4,
            [pltpu.VMEM((_KD,), jnp.int32)] * 4,
            pltpu.VMEM((_KD, _DEGW), jnp.float32),
            [pltpu.SemaphoreType.DMA] * 4,
            [pltpu.SemaphoreType.DMA] * 4,
        ],
    )
    def deg_kernel(dst_hbm, ones_hbm, zeros_hbm, out_hbm,
                   acc, stage, dbufs, idxs, ones_v, esems, ssems):
        c = lax.axis_index("c")
        s = lax.axis_index("s")
        base = c * n_half
        pltpu.sync_copy(zeros_hbm, stage)
        pltpu.sync_copy(ones_hbm, ones_v)
        pltpu.sync_copy(stage, acc.at[pl.ds(s * rpt, rpt)])
        plsc.subcore_barrier()

        for b in range(4):
            pltpu.async_copy(
                dst_hbm.at[pl.ds((s * g_cnt + b) * _KD, _KD)],
                dbufs[b], esems[b])

        def quad(t, carry):
            for b in range(4):
                g = 4 * t + b

                @pl.when(g >= 4)
                def _wait_scatter():
                    pltpu.make_async_copy(
                        ones_v, acc.at[idxs[b]], ssems[b]).wait()

                pltpu.make_async_copy(
                    dst_hbm.at[pl.ds(0, _KD)], dbufs[b], esems[b]).wait()
                for i in range(_KD // 16):
                    d = dbufs[b][pl.ds(i * 16, 16)]
                    loc = d - base
                    ok = (loc >= 0) & (loc < n_half)
                    idxs[b][pl.ds(i * 16, 16)] = jnp.where(ok, loc, n_half)
                pltpu.async_copy(ones_v, acc.at[idxs[b]], ssems[b], add=True)

                @pl.when(g + 4 < g_cnt)
                def _prefetch():
                    pltpu.async_copy(
                        dst_hbm.at[pl.ds((s * g_cnt + g + 4) * _KD, _KD)],
                        dbufs[b], esems[b])
            return carry

        lax.fori_loop(0, g_cnt // 4, quad, 0)
        for b in range(4):
            pltpu.make_async_copy(ones_v, acc.at[idxs[b]], ssems[b]).wait()
        plsc.subcore_barrier()
        pltpu.sync_copy(acc.at[pl.ds(s * rpt, rpt)], stage)
        pltpu.sync_copy(stage, out_hbm.at[pl.ds(base + s * rpt, rpt)])

    return deg_kernel


def _make_sc_aggregate(e_pad, n_half, rpt, h_dim):
    g_cnt = e_pad // (_NS * _K)
    wb = rpt // 14
    n_wb = rpt // wb
    mesh = plsc.VectorSubcoreMesh(core_axis_name="c", subcore_axis_name="s")

    assert g_cnt % 8 == 0
    t_cnt = g_cnt // 4
    ebytes = 4 * 2 * _K * 4
    rbytes = _K * h_dim * 4

    @functools.partial(
        pl.kernel,
        out_type=jax.ShapeDtypeStruct((_NC * n_half, h_dim), jnp.float32),
        mesh=mesh,
        compiler_params=pltpu.CompilerParams(use_tc_tiling_on_sc=False),
        scratch_types=[
            pltpu.VMEM_SHARED((n_half + 8, h_dim), jnp.float32),
            pltpu.VMEM((wb, h_dim), jnp.float32),
            [pltpu.VMEM((4, 2, _K), jnp.int32)] * 2,
            [pltpu.VMEM((_K,), jnp.int32)] * 2,
            [pltpu.VMEM((_K, h_dim), jnp.float32)] * 2,
            [pltpu.SemaphoreType.DMA] * 2,
            [pltpu.SemaphoreType.DMA] * 2,
            [pltpu.SemaphoreType.DMA] * 2,
        ],
    )
    def agg_kernel(rows_hbm, edges_hbm, zeros_hbm, out_hbm,
                   acc, stage, ebufs, idxs, rows, esems, gsems, ssems):
        c = lax.axis_index("c")
        s = lax.axis_index("s")
        base = c * n_half
        pltpu.sync_copy(zeros_hbm, stage)
        for k in range(n_wb):
            pltpu.sync_copy(stage, acc.at[pl.ds(s * rpt + k * wb, wb)])
        plsc.subcore_barrier()

        pltpu.async_copy(edges_hbm.at[pl.ds(s * g_cnt, 4)], ebufs[0], esems[0])

        def pair(t2, carry):
            for par in range(2):
                t = 2 * t2 + par
                eb = ebufs[par]
                pltpu.make_async_copy(
                    edges_hbm.at[pl.ds(0, 4)], eb, esems[par]).wait()
                for u in range(4):
                    g = 4 * t + u
                    b = u % 2
                    pb = 1 - b

                    @pl.when(g >= 2)
                    def _wait_scatter():
                        pltpu.make_async_copy(
                            rows[b], acc.at[idxs[b]], ssems[b]).wait()

                    _idx_from_dst(eb.at[u], idxs[b], base, n_half)
                    pltpu.async_copy(
                        rows_hbm.at[eb.at[u, 0]], rows[b], gsems[b])

                    @pl.when(g >= 1)
                    def _drain_prev():
                        pltpu.make_async_copy(
                            rows_hbm.at[eb.at[u, 0]], rows[pb],
                            gsems[pb]).wait()
                        pltpu.async_copy(
                            rows[pb], acc.at[idxs[pb]], ssems[pb], add=True)

                    if u == 0:
                        @pl.when(t + 1 < t_cnt)
                        def _prefetch():
                            pltpu.async_copy(
                                edges_hbm.at[pl.ds(s * g_cnt + 4 * (t + 1), 4)],
                                ebufs[1 - par], esems[1 - par])
            return carry

        lax.fori_loop(0, t_cnt // 2, pair, 0)
        pltpu.make_async_copy(
            rows_hbm.at[ebufs[1].at[3, 0]], rows[1], gsems[1]).wait()
        pltpu.async_copy(rows[1], acc.at[idxs[1]], ssems[1], add=True)
        for b in range(2):
            pltpu.make_async_copy(rows[b], acc.at[idxs[b]], ssems[b]).wait()
        plsc.subcore_barrier()
        for k in range(n_wb):
            off_loc = s * rpt + k * wb
            pltpu.sync_copy(acc.at[pl.ds(off_loc, wb)], stage)
            pltpu.sync_copy(stage, out_hbm.at[pl.ds(base + off_loc, wb)])

    return agg_kernel


def _prep_body(x_ref, deg_ref, we_ref, be_ref, w1_ref, hw_ref, dis_ref):
    h0 = jnp.maximum(
        jnp.dot(x_ref[...], we_ref[...], preferred_element_type=jnp.float32)
        + be_ref[...], 0.0)
    deg = deg_ref[:, 0:1] + 1.0
    dis = lax.rsqrt(jnp.maximum(deg, 1.0))
    hw = jnp.dot(h0, w1_ref[...], preferred_element_type=jnp.float32)
    hw_ref[...] = hw * dis
    dis_ref[...] = dis


def _mid_body(s_ref, hw_ref, dis_ref, b_ref, g_ref, bet_ref, w_ref, out_ref):
    dis = dis_ref[...]
    z = dis * (s_ref[...] + hw_ref[...]) + b_ref[...]
    mu = jnp.mean(z, axis=-1, keepdims=True)
    zc = z - mu
    var = jnp.mean(zc * zc, axis=-1, keepdims=True)
    h = jnp.maximum(zc * lax.rsqrt(var + EPS) * g_ref[...] + bet_ref[...], 0.0)
    out_ref[...] = jnp.dot(h, w_ref[...], preferred_element_type=jnp.float32) * dis


def _final_body(s_ref, hw_ref, dis_ref, b_ref, g_ref, bet_ref, w_ref, bsr_ref,
                out_ref):
    dis = dis_ref[...]
    z = dis * (s_ref[...] + hw_ref[...]) + b_ref[...]
    mu = jnp.mean(z, axis=-1, keepdims=True)
    zc = z - mu
    var = jnp.mean(zc * zc, axis=-1, keepdims=True)
    h = jnp.maximum(zc * lax.rsqrt(var + EPS) * g_ref[...] + bet_ref[...], 0.0)
    out_ref[...] = jnp.tanh(
        jnp.dot(h, w_ref[...], preferred_element_type=jnp.float32) + bsr_ref[...])


def _row_block(n, blk, d):
    return pl.BlockSpec((blk, d), lambda i: (i, 0))


def _whole(shape):
    return pl.BlockSpec(shape, lambda i: (0, 0))


def kernel(x, edge_index, W_enc, b_enc, W1, b1, g1, beta1, W2, b2, g2, beta2,
           W_sr, b_sr):
    n, d = x.shape
    e = edge_index.shape[1]
    h_dim = W1.shape[0]

    rpt = -(-n // (_NC * _NS * 32)) * 32
    n_half = _NS * rpt
    ept = -(-e // (_NS * 8 * _K)) * 8 * _K
    e_pad = _NS * ept
    pad = e_pad - e
    src_p = jnp.concatenate([edge_index[0], jnp.zeros((pad,), jnp.int32)])
    dst_p = jnp.concatenate(
        [edge_index[1], jnp.full((pad,), jnp.int32(1 << 20))])
    edges_p = jnp.stack(
        [src_p.reshape(-1, _K), dst_p.reshape(-1, _K)], axis=1)

    ept_d = -(-e // (_NS * 4 * _KD)) * 4 * _KD
    e_pad_d = _NS * ept_d
    dst_pd = jnp.concatenate(
        [edge_index[1], jnp.full((e_pad_d - e,), jnp.int32(1 << 20))])

    ones_deg = jnp.ones((_KD, _DEGW), jnp.float32)
    zeros_deg = jnp.zeros((rpt, _DEGW), jnp.float32)
    zeros_agg = jnp.zeros((rpt // 14, h_dim), jnp.float32)

    deg8 = _make_sc_degree(e_pad_d, n_half, rpt)(dst_pd, ones_deg, zeros_deg)

    blk = 2000
    grid = (n // blk,)
    hw1p, dis = pl.pallas_call(
        _prep_body,
        grid=grid,
        in_specs=[
            _row_block(n, blk, d),
            _row_block(n, blk, _DEGW),
            _whole((d, h_dim)),
            _whole((1, h_dim)),
            _whole((h_dim, h_dim)),
        ],
        out_specs=[_row_block(n, blk, h_dim), _row_block(n, blk, 1)],
        out_shape=[
            jax.ShapeDtypeStruct((n, h_dim), jnp.float32),
            jax.ShapeDtypeStruct((n, 1), jnp.float32),
        ],
    )(x, deg8[:n], W_enc, b_enc.reshape(1, -1), W1)

    agg = _make_sc_aggregate(e_pad, n_half, rpt, h_dim)

    s1 = agg(hw1p, edges_p, zeros_agg)
    hw2p = pl.pallas_call(
        _mid_body,
        grid=grid,
        in_specs=[
            _row_block(n, blk, h_dim),
            _row_block(n, blk, h_dim),
            _row_block(n, blk, 1),
            _whole((1, h_dim)),
            _whole((1, h_dim)),
            _whole((1, h_dim)),
            _whole((h_dim, h_dim)),
        ],
        out_specs=_row_block(n, blk, h_dim),
        out_shape=jax.ShapeDtypeStruct((n, h_dim), jnp.float32),
    )(s1[:n], hw1p, dis, b1.reshape(1, -1), g1.reshape(1, -1),
      beta1.reshape(1, -1), W2)

    s2 = agg(hw2p, edges_p, zeros_agg)
    belief = pl.pallas_call(
        _final_body,
        grid=grid,
        in_specs=[
            _row_block(n, blk, h_dim),
            _row_block(n, blk, h_dim),
            _row_block(n, blk, 1),
            _whole((1, h_dim)),
            _whole((1, h_dim)),
            _whole((1, h_dim)),
            _whole((h_dim, h_dim)),
            _whole((1, h_dim)),
        ],
        out_specs=_row_block(n, blk, h_dim),
        out_shape=jax.ShapeDtypeStruct((n, h_dim), jnp.float32),
    )(s2[:n], hw2p, dis, b2.reshape(1, -1), g2.reshape(1, -1),
      beta2.reshape(1, -1), W_sr, b_sr.reshape(1, -1))

    return belief


# speedup vs baseline: 1.3134x; 1.3134x over previous
import functools

import jax
import jax.numpy as jnp
from jax import lax
from jax.experimental import pallas as pl
from jax.experimental.pallas import tpu as pltpu
from jax.experimental.pallas import tpu_sc as plsc

EPS = 1e-5
_NC = 2
_NS = 16
_K = 128
_DEGW = 8


def _idx_from_dst(ebuf, idx, base, n_half):
    for i in range(_K // 16):
        d = ebuf[1, pl.ds(i * 16, 16)]
        loc = d - base
        ok = (loc >= 0) & (loc < n_half)
        idx[pl.ds(i * 16, 16)] = jnp.where(ok, loc, n_half)


_KD = 512


def _make_sc_degree(e_pad_deg, n_half):
    g_cnt = e_pad_deg // (_NS * _KD)
    assert g_cnt % 4 == 0 and n_half % 16 == 0
    mesh = plsc.VectorSubcoreMesh(core_axis_name="c", subcore_axis_name="s")

    @functools.partial(
        pl.kernel,
        out_type=jax.ShapeDtypeStruct((_NC * _NS, n_half), jnp.float32),
        mesh=mesh,
        compiler_params=pltpu.CompilerParams(
            use_tc_tiling_on_sc=False, needs_layout_passes=False),
        scratch_types=[
            pltpu.VMEM((n_half,), jnp.float32),
            [pltpu.VMEM((_KD,), jnp.int32)] * 4,
            [pltpu.SemaphoreType.DMA] * 4,
        ],
    )
    def deg_kernel(dst_hbm, out_hbm, hist, dbufs, esems):
        c = lax.axis_index("c")
        s = lax.axis_index("s")
        base = c * n_half
        ones16 = jnp.ones((16,), jnp.float32)

        def zero(i, carry):
            hist[pl.ds(i * 16, 16)] = jnp.zeros((16,), jnp.float32)
            return carry

        lax.fori_loop(0, n_half // 16, zero, 0)

        for b in range(4):
            pltpu.async_copy(
                dst_hbm.at[pl.ds((s * g_cnt + b) * _KD, _KD)],
                dbufs[b], esems[b])

        def quad(t, carry):
            for b in range(4):
                g = 4 * t + b
                pltpu.make_async_copy(
                    dst_hbm.at[pl.ds(0, _KD)], dbufs[b], esems[b]).wait()
                for i in range(_KD // 16):
                    d = dbufs[b][pl.ds(i * 16, 16)]
                    loc = d - base
                    ok = (loc >= 0) & (loc < n_half)
                    loc = jnp.where(ok, loc, 0)
                    plsc.addupdate_scatter(hist, [loc], ones16, mask=ok)

                @pl.when(g + 4 < g_cnt)
                def _prefetch():
                    pltpu.async_copy(
                        dst_hbm.at[pl.ds((s * g_cnt + g + 4) * _KD, _KD)],
                        dbufs[b], esems[b])
            return carry

        lax.fori_loop(0, g_cnt // 4, quad, 0)
        pltpu.sync_copy(hist, out_hbm.at[c * _NS + s])

    return deg_kernel


def _make_sc_aggregate(e_pad, n_half, rpt, h_dim):
    g_cnt = e_pad // (_NS * _K)
    wb = rpt // 14
    n_wb = rpt // wb
    mesh = plsc.VectorSubcoreMesh(core_axis_name="c", subcore_axis_name="s")

    assert g_cnt % 8 == 0
    t_cnt = g_cnt // 4
    ebytes = 4 * 2 * _K * 4
    rbytes = _K * h_dim * 4

    @functools.partial(
        pl.kernel,
        out_type=jax.ShapeDtypeStruct((_NC * n_half, h_dim), jnp.float32),
        mesh=mesh,
        compiler_params=pltpu.CompilerParams(use_tc_tiling_on_sc=False),
        scratch_types=[
            pltpu.VMEM_SHARED((n_half + 8, h_dim), jnp.float32),
            pltpu.VMEM((wb, h_dim), jnp.float32),
            [pltpu.VMEM((4, 2, _K), jnp.int32)] * 2,
            [pltpu.VMEM((_K,), jnp.int32)] * 2,
            [pltpu.VMEM((_K, h_dim), jnp.float32)] * 2,
            [pltpu.SemaphoreType.DMA] * 2,
            [pltpu.SemaphoreType.DMA] * 2,
            [pltpu.SemaphoreType.DMA] * 2,
        ],
    )
    def agg_kernel(rows_hbm, edges_hbm, zeros_hbm, out_hbm,
                   acc, stage, ebufs, idxs, rows, esems, gsems, ssems):
        c = lax.axis_index("c")
        s = lax.axis_index("s")
        base = c * n_half
        pltpu.sync_copy(zeros_hbm, stage)
        for k in range(n_wb):
            pltpu.sync_copy(stage, acc.at[pl.ds(s * rpt + k * wb, wb)])
        plsc.subcore_barrier()

        pltpu.async_copy(edges_hbm.at[pl.ds(s * g_cnt, 4)], ebufs[0], esems[0])

        def pair(t2, carry):
            for par in range(2):
                t = 2 * t2 + par
                eb = ebufs[par]
                pltpu.make_async_copy(
                    edges_hbm.at[pl.ds(0, 4)], eb, esems[par]).wait()
                for u in range(4):
                    g = 4 * t + u
                    b = u % 2
                    pb = 1 - b

                    @pl.when(g >= 2)
                    def _wait_scatter():
                        pltpu.make_async_copy(
                            rows[b], acc.at[idxs[b]], ssems[b]).wait()

                    _idx_from_dst(eb.at[u], idxs[b], base, n_half)
                    pltpu.async_copy(
                        rows_hbm.at[eb.at[u, 0]], rows[b], gsems[b])

                    @pl.when(g >= 1)
                    def _drain_prev():
                        pltpu.make_async_copy(
                            rows_hbm.at[eb.at[u, 0]], rows[pb],
                            gsems[pb]).wait()
                        pltpu.async_copy(
                            rows[pb], acc.at[idxs[pb]], ssems[pb], add=True)

                    if u == 0:
                        @pl.when(t + 1 < t_cnt)
                        def _prefetch():
                            pltpu.async_copy(
                                edges_hbm.at[pl.ds(s * g_cnt + 4 * (t + 1), 4)],
                                ebufs[1 - par], esems[1 - par])
            return carry

        lax.fori_loop(0, t_cnt // 2, pair, 0)
        pltpu.make_async_copy(
            rows_hbm.at[ebufs[1].at[3, 0]], rows[1], gsems[1]).wait()
        pltpu.async_copy(rows[1], acc.at[idxs[1]], ssems[1], add=True)
        for b in range(2):
            pltpu.make_async_copy(rows[b], acc.at[idxs[b]], ssems[b]).wait()
        plsc.subcore_barrier()
        for k in range(n_wb):
            off_loc = s * rpt + k * wb
            pltpu.sync_copy(acc.at[pl.ds(off_loc, wb)], stage)
            pltpu.sync_copy(stage, out_hbm.at[pl.ds(base + off_loc, wb)])

    return agg_kernel


def _prep_body(x_ref, hist_ref, we_ref, be_ref, w1_ref, hw_ref, dis_ref):
    h0 = jnp.maximum(
        jnp.dot(x_ref[...], we_ref[...], preferred_element_type=jnp.float32)
        + be_ref[...], 0.0)
    deg = jnp.sum(hist_ref[...], axis=0)[:, None] + 1.0
    dis = lax.rsqrt(jnp.maximum(deg, 1.0))
    hw = jnp.dot(h0, w1_ref[...], preferred_element_type=jnp.float32)
    hw_ref[...] = hw * dis
    dis_ref[...] = dis


def _mid_body(s_ref, hw_ref, dis_ref, b_ref, g_ref, bet_ref, w_ref, out_ref):
    dis = dis_ref[...]
    z = dis * (s_ref[...] + hw_ref[...]) + b_ref[...]
    mu = jnp.mean(z, axis=-1, keepdims=True)
    zc = z - mu
    var = jnp.mean(zc * zc, axis=-1, keepdims=True)
    h = jnp.maximum(zc * lax.rsqrt(var + EPS) * g_ref[...] + bet_ref[...], 0.0)
    out_ref[...] = jnp.dot(h, w_ref[...], preferred_element_type=jnp.float32) * dis


def _final_body(s_ref, hw_ref, dis_ref, b_ref, g_ref, bet_ref, w_ref, bsr_ref,
                out_ref):
    dis = dis_ref[...]
    z = dis * (s_ref[...] + hw_ref[...]) + b_ref[...]
    mu = jnp.mean(z, axis=-1, keepdims=True)
    zc = z - mu
    var = jnp.mean(zc * zc, axis=-1, keepdims=True)
    h = jnp.maximum(zc * lax.rsqrt(var + EPS) * g_ref[...] + bet_ref[...], 0.0)
    out_ref[...] = jnp.tanh(
        jnp.dot(h, w_ref[...], preferred_element_type=jnp.float32) + bsr_ref[...])


def _row_block(blk, d):
    return pl.BlockSpec((blk, d), lambda i: (i, 0))


def _whole(shape):
    return pl.BlockSpec(shape, lambda i: (0, 0))


def kernel(x, edge_index, W_enc, b_enc, W1, b1, g1, beta1, W2, b2, g2, beta2,
           W_sr, b_sr):
    n, d = x.shape
    e = edge_index.shape[1]
    h_dim = W1.shape[0]

    rpt = -(-n // (_NC * _NS * 32)) * 32
    n_half = _NS * rpt
    ept = -(-e // (_NS * 8 * _K)) * 8 * _K
    e_pad = _NS * ept
    pad = e_pad - e
    src_p = jnp.concatenate([edge_index[0], jnp.zeros((pad,), jnp.int32)])
    dst_p = jnp.concatenate(
        [edge_index[1], jnp.full((pad,), jnp.int32(1 << 20))])
    edges_p = jnp.stack(
        [src_p.reshape(-1, _K), dst_p.reshape(-1, _K)], axis=1)

    ept_d = -(-e // (_NS * 4 * _KD)) * 4 * _KD
    e_pad_d = _NS * ept_d
    dst_pd = jnp.concatenate(
        [edge_index[1], jnp.full((e_pad_d - e,), jnp.int32(1 << 20))])

    zeros_agg = jnp.zeros((rpt // 14, h_dim), jnp.float32)

    hist = _make_sc_degree(e_pad_d, n_half)(dst_pd)

    n_pad = _NC * n_half
    blk = 1792
    assert n_half % blk == 0 and blk % 128 == 0
    nbh = n_half // blk
    grid = (n_pad // blk,)
    x_p = jnp.concatenate([x, jnp.zeros((n_pad - n, d), jnp.float32)])

    hw1p, dis = pl.pallas_call(
        _prep_body,
        grid=grid,
        in_specs=[
            _row_block(blk, d),
            pl.BlockSpec((_NS, blk), lambda i: (i // nbh, i % nbh)),
            _whole((d, h_dim)),
            _whole((1, h_dim)),
            _whole((h_dim, h_dim)),
        ],
        out_specs=[_row_block(blk, h_dim), _row_block(blk, 1)],
        out_shape=[
            jax.ShapeDtypeStruct((n_pad, h_dim), jnp.float32),
            jax.ShapeDtypeStruct((n_pad, 1), jnp.float32),
        ],
    )(x_p, hist, W_enc, b_enc.reshape(1, -1), W1)

    agg = _make_sc_aggregate(e_pad, n_half, rpt, h_dim)

    s1 = agg(hw1p, edges_p, zeros_agg)
    hw2p = pl.pallas_call(
        _mid_body,
        grid=grid,
        in_specs=[
            _row_block(blk, h_dim),
            _row_block(blk, h_dim),
            _row_block(blk, 1),
            _whole((1, h_dim)),
            _whole((1, h_dim)),
            _whole((1, h_dim)),
            _whole((h_dim, h_dim)),
        ],
        out_specs=_row_block(blk, h_dim),
        out_shape=jax.ShapeDtypeStruct((n_pad, h_dim), jnp.float32),
    )(s1, hw1p, dis, b1.reshape(1, -1), g1.reshape(1, -1),
      beta1.reshape(1, -1), W2)

    s2 = agg(hw2p, edges_p, zeros_agg)
    belief = pl.pallas_call(
        _final_body,
        grid=grid,
        in_specs=[
            _row_block(blk, h_dim),
            _row_block(blk, h_dim),
            _row_block(blk, 1),
            _whole((1, h_dim)),
            _whole((1, h_dim)),
            _whole((1, h_dim)),
            _whole((h_dim, h_dim)),
            _whole((1, h_dim)),
        ],
        out_specs=_row_block(blk, h_dim),
        out_shape=jax.ShapeDtypeStruct((n_pad, h_dim), jnp.float32),
    )(s2, hw2p, dis, b2.reshape(1, -1), g2.reshape(1, -1),
      beta2.reshape(1, -1), W_sr, b_sr.reshape(1, -1))

    return belief[:n]


# speedup vs baseline: 2.6982x; 2.0544x over previous
import functools

import jax
import jax.numpy as jnp
from jax import lax
from jax.experimental import pallas as pl
from jax.experimental.pallas import tpu as pltpu
from jax.experimental.pallas import tpu_sc as plsc

EPS = 1e-5
_NC = 2
_NS = 16
_K = 128
_DEGW = 8


_KD = 512


def _make_sc_degree(e_pad_deg, n_half):
    g_cnt = e_pad_deg // (_NS * _KD)
    assert g_cnt % 4 == 0 and n_half % 16 == 0
    mesh = plsc.VectorSubcoreMesh(core_axis_name="c", subcore_axis_name="s")

    @functools.partial(
        pl.kernel,
        out_type=jax.ShapeDtypeStruct((_NC * _NS, n_half), jnp.float32),
        mesh=mesh,
        compiler_params=pltpu.CompilerParams(
            use_tc_tiling_on_sc=False, needs_layout_passes=False),
        scratch_types=[
            pltpu.VMEM((n_half,), jnp.float32),
            [pltpu.VMEM((_KD,), jnp.int32)] * 4,
            [pltpu.SemaphoreType.DMA] * 4,
        ],
    )
    def deg_kernel(dst_hbm, out_hbm, hist, dbufs, esems):
        c = lax.axis_index("c")
        s = lax.axis_index("s")
        base = c * n_half
        ones16 = jnp.ones((16,), jnp.float32)

        def zero(i, carry):
            hist[pl.ds(i * 16, 16)] = jnp.zeros((16,), jnp.float32)
            return carry

        lax.fori_loop(0, n_half // 16, zero, 0)

        for b in range(4):
            pltpu.async_copy(
                dst_hbm.at[pl.ds((s * g_cnt + b) * _KD, _KD)],
                dbufs[b], esems[b])

        def quad(t, carry):
            for b in range(4):
                g = 4 * t + b
                pltpu.make_async_copy(
                    dst_hbm.at[pl.ds(0, _KD)], dbufs[b], esems[b]).wait()
                for i in range(_KD // 16):
                    d = dbufs[b][pl.ds(i * 16, 16)]
                    loc = d - base
                    ok = (loc >= 0) & (loc < n_half)
                    loc = jnp.where(ok, loc, 0)
                    plsc.addupdate_scatter(hist, [loc], ones16, mask=ok)

                @pl.when(g + 4 < g_cnt)
                def _prefetch():
                    pltpu.async_copy(
                        dst_hbm.at[pl.ds((s * g_cnt + g + 4) * _KD, _KD)],
                        dbufs[b], esems[b])
            return carry

        lax.fori_loop(0, g_cnt // 4, quad, 0)
        pltpu.sync_copy(hist, out_hbm.at[c * _NS + s])

    return deg_kernel


_F = 512
_MAXC = 400


def _make_sc_partition(e_pad, n_half):
    g_cnt = e_pad // (_NS * _K)
    assert g_cnt % 8 == 0
    t_cnt = g_cnt // 4
    mesh = plsc.VectorSubcoreMesh(core_axis_name="c", subcore_axis_name="s")

    @functools.partial(
        pl.kernel,
        out_type=[
            jax.ShapeDtypeStruct((_NS, _NC, _MAXC, 2, _K), jnp.int32),
            jax.ShapeDtypeStruct((_NS * _NC, 16), jnp.int32),
        ],
        mesh=mesh,
        compiler_params=pltpu.CompilerParams(
            use_tc_tiling_on_sc=False, needs_layout_passes=False),
        scratch_types=[
            [pltpu.VMEM((4, 2, _K), jnp.int32)] * 2,
            pltpu.VMEM((_F + 2 * _K + 16,), jnp.int32),
            pltpu.VMEM((_F + 2 * _K + 16,), jnp.int32),
            pltpu.VMEM((5, 2, _K), jnp.int32),
            pltpu.VMEM((16,), jnp.int32),
            [pltpu.SemaphoreType.DMA] * 2,
            pltpu.SemaphoreType.DMA,
        ],
    )
    def part_kernel(edges_hbm, out_hbm, cnt_hbm,
                    ebufs, fsrc, fdst, fbuf, cbuf, esems, fsem):
        c = lax.axis_index("c")
        s = lax.axis_index("s")
        base = c * n_half
        pltpu.async_copy(edges_hbm.at[pl.ds(s * g_cnt, 4)], ebufs[0], esems[0])

        def do_flush(carry):
            cnt, nf = carry

            @pl.when(nf > 0)
            def _wait_prev():
                pltpu.make_async_copy(
                    fbuf.at[pl.ds(0, 4)], out_hbm.at[s, c, pl.ds(0, 4)],
                    fsem).wait()

            for q in range(4):
                for i in range(_K // 16):
                    fbuf[q, 0, pl.ds(i * 16, 16)] = (
                        fsrc[pl.ds(q * _K + i * 16, 16)])
                    fbuf[q, 1, pl.ds(i * 16, 16)] = (
                        fdst[pl.ds(q * _K + i * 16, 16)])
            pltpu.async_copy(
                fbuf.at[pl.ds(0, 4)], out_hbm.at[s, c, pl.ds(nf * 4, 4)],
                fsem)
            for i in range((2 * _K + 16) // 16):
                v0 = fsrc[pl.ds(_F + i * 16, 16)]
                fsrc[pl.ds(i * 16, 16)] = v0
                v1 = fdst[pl.ds(_F + i * 16, 16)]
                fdst[pl.ds(i * 16, 16)] = v1
            return cnt - _F, nf + 1

        def group(t2, carry):
            for par in range(2):
                t = 2 * t2 + par
                eb = ebufs[par]
                pltpu.make_async_copy(
                    edges_hbm.at[pl.ds(0, 4)], eb, esems[par]).wait()

                @pl.when(t + 1 < t_cnt)
                def _prefetch():
                    pltpu.async_copy(
                        edges_hbm.at[pl.ds(s * g_cnt + 4 * (t + 1), 4)],
                        ebufs[1 - par], esems[1 - par])

                for u in range(4):
                    cnt, nf = carry
                    for i in range(_K // 16):
                        src16 = eb[u, 0, pl.ds(i * 16, 16)]
                        d16 = eb[u, 1, pl.ds(i * 16, 16)]
                        loc = d16 - base
                        ok = (loc >= 0) & (loc < n_half)
                        plsc.store_compressed(
                            fsrc.at[pl.ds(cnt, 16)], src16, mask=ok)
                        plsc.store_compressed(
                            fdst.at[pl.ds(cnt, 16)], loc, mask=ok)
                        cnt = cnt + jnp.sum(ok.astype(jnp.int32))
                    carry = lax.cond(cnt >= _F, do_flush, lambda a: a,
                                     (cnt, nf))
            return carry

        cnt, nf = lax.fori_loop(0, t_cnt // 2, group, (0, 0))

        iota = lax.iota(jnp.int32, 16)
        for k in range(5 * _K // 16):
            pos = k * 16 + iota
            m = pos < cnt
            fsrc[pl.ds(k * 16, 16)] = jnp.where(
                m, fsrc[pl.ds(k * 16, 16)], 0)
            fdst[pl.ds(k * 16, 16)] = jnp.where(
                m, fdst[pl.ds(k * 16, 16)], n_half)
        @pl.when(nf > 0)
        def _wait_prev():
            pltpu.make_async_copy(
                fbuf.at[pl.ds(0, 4)], out_hbm.at[s, c, pl.ds(0, 4)],
                fsem).wait()

        for q in range(5):
            for i in range(_K // 16):
                fbuf[q, 0, pl.ds(i * 16, 16)] = (
                    fsrc[pl.ds(q * _K + i * 16, 16)])
                fbuf[q, 1, pl.ds(i * 16, 16)] = (
                    fdst[pl.ds(q * _K + i * 16, 16)])

        pltpu.async_copy(fbuf, out_hbm.at[s, c, pl.ds(nf * 4, 5)], fsem)
        cbuf[pl.ds(0, 16)] = jnp.where(iota == 0, nf * _F + cnt, 0)
        pltpu.sync_copy(cbuf, cnt_hbm.at[s * _NC + c])
        pltpu.make_async_copy(
            fbuf, out_hbm.at[s, c, pl.ds(0, 5)], fsem).wait()

    return part_kernel


def _make_sc_aggregate(n_half, rpt, h_dim):
    wb = rpt // 14
    n_wb = rpt // wb
    mesh = plsc.VectorSubcoreMesh(core_axis_name="c", subcore_axis_name="s")

    @functools.partial(
        pl.kernel,
        out_type=jax.ShapeDtypeStruct((_NC * n_half, h_dim), jnp.float32),
        mesh=mesh,
        compiler_params=pltpu.CompilerParams(use_tc_tiling_on_sc=False),
        scratch_types=[
            pltpu.VMEM_SHARED((n_half + 8, h_dim), jnp.float32),
            pltpu.VMEM((wb, h_dim), jnp.float32),
            [pltpu.VMEM((2, _K), jnp.int32)] * 4,
            [pltpu.VMEM((_K, h_dim), jnp.float32)] * 2,
            pltpu.VMEM((16,), jnp.int32),
            [pltpu.SemaphoreType.DMA] * 4,
            [pltpu.SemaphoreType.DMA] * 2,
            [pltpu.SemaphoreType.DMA] * 2,
        ],
    )
    def agg_kernel(rows_hbm, edges_hbm, cnt_hbm, zeros_hbm, out_hbm,
                   acc, stage, ebufs, rows, cbuf, esems, gsems, ssems):
        c = lax.axis_index("c")
        s = lax.axis_index("s")
        base = c * n_half
        pltpu.sync_copy(cnt_hbm.at[s * _NC + c], cbuf)
        cv = cbuf[pl.ds(0, 16)]
        t_dyn = (cv[0] + _K - 1) // _K
        pltpu.sync_copy(zeros_hbm, stage)
        for k in range(n_wb):
            pltpu.sync_copy(stage, acc.at[pl.ds(s * rpt + k * wb, wb)])
        plsc.subcore_barrier()

        for b in range(2):
            @pl.when(b < t_dyn)
            def _pro():
                pltpu.async_copy(edges_hbm.at[s, c, b], ebufs[b], esems[b])

        def quad(t, carry):
            for u in range(4):
                g4 = 4 * t + u
                b2 = u % 2
                p2 = 1 - b2

                @pl.when(g4 < t_dyn)
                def _produce():
                    @pl.when(g4 >= 2)
                    def _wait_s():
                        pltpu.make_async_copy(
                            rows[b2], acc.at[ebufs[u].at[1]],
                            ssems[b2]).wait()

                    @pl.when(g4 + 2 < t_dyn)
                    def _pref():
                        pltpu.async_copy(
                            edges_hbm.at[s, c, g4 + 2],
                            ebufs[(u + 2) % 4], esems[(u + 2) % 4])

                    pltpu.make_async_copy(
                        edges_hbm.at[s, c, 0], ebufs[u], esems[u]).wait()
                    pltpu.async_copy(
                        rows_hbm.at[ebufs[u].at[0]], rows[b2], gsems[b2])

                @pl.when((g4 >= 1) & (g4 - 1 < t_dyn))
                def _drain():
                    pltpu.make_async_copy(
                        rows_hbm.at[ebufs[(u + 3) % 4].at[0]], rows[p2],
                        gsems[p2]).wait()
                    pltpu.async_copy(
                        rows[p2], acc.at[ebufs[(u + 3) % 4].at[1]],
                        ssems[p2], add=True)
            return carry

        lax.fori_loop(0, _MAXC // 4 + 1, quad, 0)

        @pl.when(t_dyn >= 1)
        def _fin0():
            pltpu.make_async_copy(
                rows[0], acc.at[ebufs[0].at[1]], ssems[0]).wait()

        @pl.when(t_dyn >= 2)
        def _fin1():
            pltpu.make_async_copy(
                rows[1], acc.at[ebufs[1].at[1]], ssems[1]).wait()

        plsc.subcore_barrier()
        for k in range(n_wb):
            off_loc = s * rpt + k * wb
            pltpu.sync_copy(acc.at[pl.ds(off_loc, wb)], stage)
            pltpu.sync_copy(stage, out_hbm.at[pl.ds(base + off_loc, wb)])

    return agg_kernel


def _prep_body(x_ref, hist_ref, we_ref, be_ref, w1_ref, hw_ref, dis_ref):
    h0 = jnp.maximum(
        jnp.dot(x_ref[...], we_ref[...], preferred_element_type=jnp.float32)
        + be_ref[...], 0.0)
    deg = jnp.sum(hist_ref[...], axis=0)[:, None] + 1.0
    dis = lax.rsqrt(jnp.maximum(deg, 1.0))
    hw = jnp.dot(h0, w1_ref[...], preferred_element_type=jnp.float32)
    hw_ref[...] = hw * dis
    dis_ref[...] = dis


def _mid_body(s_ref, hw_ref, dis_ref, b_ref, g_ref, bet_ref, w_ref, out_ref):
    dis = dis_ref[...]
    z = dis * (s_ref[...] + hw_ref[...]) + b_ref[...]
    mu = jnp.mean(z, axis=-1, keepdims=True)
    zc = z - mu
    var = jnp.mean(zc * zc, axis=-1, keepdims=True)
    h = jnp.maximum(zc * lax.rsqrt(var + EPS) * g_ref[...] + bet_ref[...], 0.0)
    out_ref[...] = jnp.dot(h, w_ref[...], preferred_element_type=jnp.float32) * dis


def _final_body(s_ref, hw_ref, dis_ref, b_ref, g_ref, bet_ref, w_ref, bsr_ref,
                out_ref):
    dis = dis_ref[...]
    z = dis * (s_ref[...] + hw_ref[...]) + b_ref[...]
    mu = jnp.mean(z, axis=-1, keepdims=True)
    zc = z - mu
    var = jnp.mean(zc * zc, axis=-1, keepdims=True)
    h = jnp.maximum(zc * lax.rsqrt(var + EPS) * g_ref[...] + bet_ref[...], 0.0)
    out_ref[...] = jnp.tanh(
        jnp.dot(h, w_ref[...], preferred_element_type=jnp.float32) + bsr_ref[...])


def _row_block(blk, d):
    return pl.BlockSpec((blk, d), lambda i: (i, 0))


def _whole(shape):
    return pl.BlockSpec(shape, lambda i: (0, 0))


def kernel(x, edge_index, W_enc, b_enc, W1, b1, g1, beta1, W2, b2, g2, beta2,
           W_sr, b_sr):
    n, d = x.shape
    e = edge_index.shape[1]
    h_dim = W1.shape[0]

    rpt = -(-n // (_NC * _NS * 32)) * 32
    n_half = _NS * rpt
    ept = -(-e // (_NS * 8 * _K)) * 8 * _K
    e_pad = _NS * ept
    pad = e_pad - e
    src_p = jnp.concatenate([edge_index[0], jnp.zeros((pad,), jnp.int32)])
    dst_p = jnp.concatenate(
        [edge_index[1], jnp.full((pad,), jnp.int32(1 << 20))])
    edges_p = jnp.stack(
        [src_p.reshape(-1, _K), dst_p.reshape(-1, _K)], axis=1)

    ept_d = -(-e // (_NS * 4 * _KD)) * 4 * _KD
    e_pad_d = _NS * ept_d
    dst_pd = jnp.concatenate(
        [edge_index[1], jnp.full((e_pad_d - e,), jnp.int32(1 << 20))])

    zeros_agg = jnp.zeros((rpt // 14, h_dim), jnp.float32)

    hist = _make_sc_degree(e_pad_d, n_half)(dst_pd)

    n_pad = _NC * n_half
    blk = 1792
    assert n_half % blk == 0 and blk % 128 == 0
    nbh = n_half // blk
    grid = (n_pad // blk,)
    x_p = jnp.concatenate([x, jnp.zeros((n_pad - n, d), jnp.float32)])

    hw1p, dis = pl.pallas_call(
        _prep_body,
        grid=grid,
        in_specs=[
            _row_block(blk, d),
            pl.BlockSpec((_NS, blk), lambda i: (i // nbh, i % nbh)),
            _whole((d, h_dim)),
            _whole((1, h_dim)),
            _whole((h_dim, h_dim)),
        ],
        out_specs=[_row_block(blk, h_dim), _row_block(blk, 1)],
        out_shape=[
            jax.ShapeDtypeStruct((n_pad, h_dim), jnp.float32),
            jax.ShapeDtypeStruct((n_pad, 1), jnp.float32),
        ],
    )(x_p, hist, W_enc, b_enc.reshape(1, -1), W1)

    part_edges, part_cnts = _make_sc_partition(e_pad, n_half)(edges_p)
    agg = _make_sc_aggregate(n_half, rpt, h_dim)

    s1 = agg(hw1p, part_edges, part_cnts, zeros_agg)
    hw2p = pl.pallas_call(
        _mid_body,
        grid=grid,
        in_specs=[
            _row_block(blk, h_dim),
            _row_block(blk, h_dim),
            _row_block(blk, 1),
            _whole((1, h_dim)),
            _whole((1, h_dim)),
            _whole((1, h_dim)),
            _whole((h_dim, h_dim)),
        ],
        out_specs=_row_block(blk, h_dim),
        out_shape=jax.ShapeDtypeStruct((n_pad, h_dim), jnp.float32),
    )(s1, hw1p, dis, b1.reshape(1, -1), g1.reshape(1, -1),
      beta1.reshape(1, -1), W2)

    s2 = agg(hw2p, part_edges, part_cnts, zeros_agg)
    belief = pl.pallas_call(
        _final_body,
        grid=grid,
        in_specs=[
            _row_block(blk, h_dim),
            _row_block(blk, h_dim),
            _row_block(blk, 1),
            _whole((1, h_dim)),
            _whole((1, h_dim)),
            _whole((1, h_dim)),
            _whole((h_dim, h_dim)),
            _whole((1, h_dim)),
        ],
        out_specs=_row_block(blk, h_dim),
        out_shape=jax.ShapeDtypeStruct((n_pad, h_dim), jnp.float32),
    )(s2, hw2p, dis, b2.reshape(1, -1), g2.reshape(1, -1),
      beta2.reshape(1, -1), W_sr, b_sr.reshape(1, -1))

    return belief[:n]
